# Initial kernel scaffold; baseline (speedup 1.0000x reference)
import jax
import jax.numpy as jnp
from jax.experimental import pallas as pl


def kernel(x, pe_table):
    raise NotImplementedError("write your pallas kernel here")


# speedup vs baseline: 2.1458x; 2.1458x over previous
import jax
import jax.numpy as jnp
from jax.experimental import pallas as pl
from jax.experimental.pallas import tpu as pltpu


def _body(x_ref, pe_ref, o_ref):
    pe_t = pe_ref[...].T
    o_ref[...] = x_ref[...] + pe_t[None]


def kernel(x, pe_table):
    B, D, S = x.shape
    s_blk = 512 if S % 512 == 0 else S
    return pl.pallas_call(
        _body,
        grid=(S // s_blk, B),
        in_specs=[
            pl.BlockSpec((1, D, s_blk), lambda i, j: (j, 0, i)),
            pl.BlockSpec((s_blk, D), lambda i, j: (i, 0)),
        ],
        out_specs=pl.BlockSpec((1, D, s_blk), lambda i, j: (j, 0, i)),
        out_shape=jax.ShapeDtypeStruct(x.shape, x.dtype),
    )(x, pe_table)
